# baseline (speedup 1.0000x reference)
import jax
import jax.numpy as jnp
from jax.experimental import pallas as pl
from jax.experimental.pallas import tpu as pltpu

_NUM_BINS = 100000
_SALT_ADD = (42 * 0x9E3779B9) & 0xFFFFFFFF

_GRID = 32


def _hash_block(x_ref, o_ref):
    z = x_ref[...].astype(jnp.uint32)
    z = z + jnp.uint32(_SALT_ADD)
    z = (z ^ (z >> 16)) * jnp.uint32(0x85EBCA6B)
    z = (z ^ (z >> 13)) * jnp.uint32(0xC2B2AE35)
    z = z ^ (z >> 16)
    q = z // jnp.uint32(_NUM_BINS)
    r = z - q * jnp.uint32(_NUM_BINS)
    o_ref[...] = r.astype(jnp.int32)


def _pipelined_body(m, n):
    bc = n // _GRID

    def body(x_hbm, o_hbm):
        pltpu.emit_pipeline(
            _hash_block,
            grid=(_GRID,),
            in_specs=[pl.BlockSpec((m, bc), lambda i: (0, i))],
            out_specs=[pl.BlockSpec((m, bc), lambda i: (0, i))],
        )(x_hbm, o_hbm)

    return body


def kernel(inputs):
    n, m = inputs.shape
    xt = jnp.swapaxes(inputs, 0, 1)
    xt = pltpu.with_memory_space_constraint(xt, pltpu.MemorySpace.HBM)
    out_t = pl.pallas_call(
        _pipelined_body(m, n),
        in_specs=[pl.BlockSpec(memory_space=pltpu.MemorySpace.HBM)],
        out_specs=pl.BlockSpec(memory_space=pltpu.MemorySpace.HBM),
        out_shape=jax.ShapeDtypeStruct((m, n), jnp.int32),
    )(xt)
    return jnp.swapaxes(out_t, 0, 1)


# speedup vs baseline: 1.1575x; 1.1575x over previous
import jax
import jax.numpy as jnp
from jax.experimental import pallas as pl
from jax.experimental.pallas import tpu as pltpu

_NUM_BINS = 100000
_SALT_ADD = (42 * 0x9E3779B9) & 0xFFFFFFFF


def _hash_block(x_ref, o_ref):
    z = x_ref[...].astype(jnp.uint32)
    z = z + jnp.uint32(_SALT_ADD)
    z = (z ^ (z >> 16)) * jnp.uint32(0x85EBCA6B)
    z = (z ^ (z >> 13)) * jnp.uint32(0xC2B2AE35)
    z = z ^ (z >> 16)
    q = z // jnp.uint32(_NUM_BINS)
    r = z - q * jnp.uint32(_NUM_BINS)
    o_ref[...] = r.astype(jnp.int32)


def kernel(inputs):
    n, m = inputs.shape
    xt = jnp.swapaxes(inputs, 0, 1)
    xt = pltpu.with_memory_space_constraint(xt, pltpu.MemorySpace.HBM)
    grid = m // 8
    out_t = pl.pallas_call(
        _hash_block,
        grid=(grid,),
        in_specs=[pl.BlockSpec((8, n), lambda i: (i, 0))],
        out_specs=pl.BlockSpec((8, n), lambda i: (i, 0)),
        out_shape=jax.ShapeDtypeStruct((m, n), jnp.int32),
    )(xt)
    return jnp.swapaxes(out_t, 0, 1)


# speedup vs baseline: 1.8753x; 1.6202x over previous
import jax
import jax.numpy as jnp
from jax.experimental import pallas as pl
from jax.experimental.pallas import tpu as pltpu

_NUM_BINS = 100000
_SALT_ADD = (42 * 0x9E3779B9) & 0xFFFFFFFF


def _hash_block(x_ref, o_ref):
    z = x_ref[...].astype(jnp.uint32)
    z = z + jnp.uint32(_SALT_ADD)
    z = (z ^ (z >> 16)) * jnp.uint32(0x85EBCA6B)
    z = (z ^ (z >> 13)) * jnp.uint32(0xC2B2AE35)
    z = z ^ (z >> 16)
    q = z // jnp.uint32(_NUM_BINS)
    r = z - q * jnp.uint32(_NUM_BINS)
    o_ref[...] = r.astype(jnp.int32)


def kernel(inputs):
    n, m = inputs.shape
    xt = jnp.swapaxes(inputs, 0, 1)
    xt = pltpu.with_memory_space_constraint(xt, pltpu.MemorySpace.HBM)
    grid = 5
    br = m // grid
    out_t = pl.pallas_call(
        _hash_block,
        grid=(grid,),
        in_specs=[pl.BlockSpec((br, n), lambda i: (i, 0))],
        out_specs=pl.BlockSpec((br, n), lambda i: (i, 0)),
        out_shape=jax.ShapeDtypeStruct((m, n), jnp.int32),
    )(xt)
    return jnp.swapaxes(out_t, 0, 1)
